# Initial kernel scaffold; baseline (speedup 1.0000x reference)
import jax
import jax.numpy as jnp
from jax.experimental import pallas as pl


def kernel(x, edge_index, position_bias, W_gat, a_src, a_dst, W_fuse, b_fuse, diffusion_weight):
    raise NotImplementedError("write your pallas kernel here")


# speedup vs baseline: 191.8967x; 191.8967x over previous
import dataclasses
import functools

import jax
import jax.numpy as jnp
from jax import lax
from jax.experimental import pallas as pl
from jax.experimental.pallas import tpu as pltpu
from jax.experimental.pallas import tpu_sc as plsc

H = 8
DH = 16
NUM_HOPS = 8
N = 10000
E = 320000
D = 128

NC = 2
NS = 16
NW = NC * NS
NP = 10240
RP = NP // NS
CH = 80
NCH_AB = E // (NW * CH)
NCH_C = E // (NS * CH)

_mesh = plsc.VectorSubcoreMesh(core_axis_name="c", subcore_axis_name="s")


def _sc_params():
    cp = pltpu.CompilerParams()
    fields = pltpu.CompilerParams.__dataclass_fields__
    if "needs_layout_passes" in fields:
        cp = dataclasses.replace(cp, needs_layout_passes=False)
    if "use_tc_tiling_on_sc" in fields:
        cp = dataclasses.replace(cp, use_tc_tiling_on_sc=False)
    return cp


def _f32(shape):
    return jax.ShapeDtypeStruct(shape, jnp.float32)


def _tc1_body(x_ref, w_ref, as_ref, ad_ref, xw_ref, as16_ref, ad16_ref):
    xw = jnp.dot(x_ref[...], w_ref[...], preferred_element_type=jnp.float32)
    xw_ref[...] = xw
    as16_ref[...] = jnp.dot(xw, as_ref[...], preferred_element_type=jnp.float32)
    ad16_ref[...] = jnp.dot(xw, ad_ref[...], preferred_element_type=jnp.float32)


def _tc1(x, w_gat, a_s16, a_d16):
    blk = 1000
    grid = (N // blk,)
    return pl.pallas_call(
        _tc1_body,
        grid=grid,
        in_specs=[
            pl.BlockSpec((blk, D), lambda i: (i, 0)),
            pl.BlockSpec((D, D), lambda i: (0, 0)),
            pl.BlockSpec((D, DH), lambda i: (0, 0)),
            pl.BlockSpec((D, DH), lambda i: (0, 0)),
        ],
        out_specs=[
            pl.BlockSpec((blk, D), lambda i: (i, 0)),
            pl.BlockSpec((blk, DH), lambda i: (i, 0)),
            pl.BlockSpec((blk, DH), lambda i: (i, 0)),
        ],
        out_shape=[_f32((N, D)), _f32((N, DH)), _f32((N, DH))],
    )(x, w_gat, a_s16, a_d16)


def _tc2_body(x_ref, a0_ref, a1_ref, wt_ref, wb_ref, b_ref, m16_ref, st_ref):
    agg = a0_ref[...] + a1_ref[...]
    f = (jnp.dot(x_ref[...], wt_ref[...], preferred_element_type=jnp.float32)
         + jnp.dot(agg, wb_ref[...], preferred_element_type=jnp.float32)
         + b_ref[...])
    f = jnp.maximum(f, 0.0)
    st = jnp.dot(f, m16_ref[...], preferred_element_type=jnp.float32)
    row = lax.broadcasted_iota(jnp.int32, st.shape, 0)
    lane = lax.broadcasted_iota(jnp.int32, st.shape, 1)
    first = (pl.program_id(0) == 0) & (row == 0) & (lane == H)
    st_ref[...] = jnp.where(first, 1.0, st)


def _tc2(x, agg0, agg1, wf_top, wf_bot, b2d, m16):
    blk = 1000
    grid = (N // blk,)
    return pl.pallas_call(
        _tc2_body,
        grid=grid,
        in_specs=[
            pl.BlockSpec((blk, D), lambda i: (i, 0)),
            pl.BlockSpec((blk, D), lambda i: (i, 0)),
            pl.BlockSpec((blk, D), lambda i: (i, 0)),
            pl.BlockSpec((D, D), lambda i: (0, 0)),
            pl.BlockSpec((D, D), lambda i: (0, 0)),
            pl.BlockSpec((1, D), lambda i: (0, 0)),
            pl.BlockSpec((D, DH), lambda i: (0, 0)),
        ],
        out_specs=pl.BlockSpec((blk, DH), lambda i: (i, 0)),
        out_shape=_f32((N, DH)),
    )(x, agg0, agg1, wf_top, wf_bot, b2d, m16)


def _scA_body(as_hbm, ad_hbm, src_hbm, dst_hbm, ex_hbm, dnp_hbm,
              src_v, dst_v, ga_v, gb_v, acc_v, t_v, acc_sh):
    c = lax.axis_index("c")
    s = lax.axis_index("s")
    base = s * RP
    zero = jnp.zeros((DH,), jnp.float32)

    @pl.loop(0, RP)
    def _(r):
        t_v[r] = zero
    pltpu.sync_copy(t_v, acc_sh.at[pl.ds(base, RP)])
    pltpu.sync_copy(src_hbm.at[c, s], src_v)
    pltpu.sync_copy(dst_hbm.at[c, s], dst_v)
    plsc.subcore_barrier()

    @pl.loop(0, NCH_AB)
    def _(j):
        pltpu.sync_copy(as_hbm.at[src_v.at[j]], ga_v)
        pltpu.sync_copy(ad_hbm.at[dst_v.at[j]], gb_v)

        @pl.loop(0, CH)
        def _(i):
            v = ga_v[i] + gb_v[i]
            v = jnp.where(v > 0.0, v, 0.2 * v)
            acc_v[i] = jnp.exp(v)

        pltpu.sync_copy(acc_v, ex_hbm.at[c, s, j])
        pltpu.sync_copy(acc_v, acc_sh.at[dst_v.at[j]], add=True)

    plsc.subcore_barrier()
    pltpu.sync_copy(acc_sh.at[pl.ds(base, RP)], t_v)
    pltpu.sync_copy(t_v, dnp_hbm.at[c, pl.ds(base, RP)])


def _scA(as16, ad16, src_ab, dst_ab):
    k = pl.kernel(
        _scA_body,
        out_type=(_f32((NC, NS, NCH_AB, CH, DH)), _f32((NC, NP, DH))),
        mesh=_mesh,
        scratch_types=[
            pltpu.VMEM((NCH_AB, CH), jnp.int32),
            pltpu.VMEM((NCH_AB, CH), jnp.int32),
            pltpu.VMEM((CH, DH), jnp.float32),
            pltpu.VMEM((CH, DH), jnp.float32),
            pltpu.VMEM((CH, DH), jnp.float32),
            pltpu.VMEM((RP, DH), jnp.float32),
            pltpu.VMEM_SHARED((NP, DH), jnp.float32),
        ],
        compiler_params=_sc_params(),
    )
    return k(as16, ad16, src_ab, dst_ab)


def _scA2_body(dnp_hbm, rcp_hbm, inv_hbm, d0_v, d1_v, rcp_v, inv_v):
    c = lax.axis_index("c")
    s = lax.axis_index("s")

    @pl.when(c == 0)
    def _():
        base = s * RP
        pltpu.sync_copy(dnp_hbm.at[0, pl.ds(base, RP)], d0_v)
        pltpu.sync_copy(dnp_hbm.at[1, pl.ds(base, RP)], d1_v)
        lane = lax.iota(jnp.int32, DH)
        is_den = lane < H

        @pl.loop(0, RP)
        def _(r):
            drow = d0_v[r] + d1_v[r]
            d0_v[r] = drow
            degv = plsc.load_gather(
                d0_v, [jnp.full((DH,), r, jnp.int32), jnp.full((DH,), H, jnp.int32)])
            degc = jnp.maximum(degv, 1.0)
            rcp_v[r] = 1.0 / jnp.where(is_den, drow + 1e-16, degc)
            inv_v[r] = 1.0 / degc

        pltpu.sync_copy(rcp_v, rcp_hbm.at[pl.ds(base, RP)])
        pltpu.sync_copy(inv_v, inv_hbm.at[pl.ds(base, RP)])


def _scA2(dnp):
    k = pl.kernel(
        _scA2_body,
        out_type=(_f32((NP, DH)), _f32((NP, DH))),
        mesh=_mesh,
        scratch_types=[
            pltpu.VMEM((RP, DH), jnp.float32),
            pltpu.VMEM((RP, DH), jnp.float32),
            pltpu.VMEM((RP, DH), jnp.float32),
            pltpu.VMEM((RP, DH), jnp.float32),
        ],
        compiler_params=_sc_params(),
    )
    return k(dnp)


def _scB_body(xw_hbm, ex_hbm, rcp_hbm, src_hbm, dst_hbm, agg_hbm,
              src_v, dst_v, xw_v, rcp_v, ex_v, al_v, z_v, agg_sh):
    c = lax.axis_index("c")
    s = lax.axis_index("s")
    base = s * RP
    zero = jnp.zeros((DH,), jnp.float32)

    @pl.loop(0, 64)
    def _(r):
        @pl.loop(0, D // DH)
        def _(q):
            z_v[r, pl.ds(q * DH, DH)] = zero

    @pl.loop(0, RP // 64)
    def _(p):
        pltpu.sync_copy(z_v, agg_sh.at[pl.ds(base + p * 64, 64)])
    pltpu.sync_copy(src_hbm.at[c, s], src_v)
    pltpu.sync_copy(dst_hbm.at[c, s], dst_v)
    plsc.subcore_barrier()

    @pl.loop(0, NCH_AB)
    def _(j):
        pltpu.sync_copy(xw_hbm.at[src_v.at[j]], xw_v)
        pltpu.sync_copy(rcp_hbm.at[dst_v.at[j]], rcp_v)
        pltpu.sync_copy(ex_hbm.at[c, s, j], ex_v)

        @pl.loop(0, CH)
        def _(e):
            al_v[e] = ex_v[e] * rcp_v[e]
            for h in range(H):
                bc = plsc.load_gather(
                    al_v, [jnp.full((DH,), e, jnp.int32),
                           jnp.full((DH,), h, jnp.int32)])
                xw_v[e, pl.ds(h * DH, DH)] = xw_v[e, pl.ds(h * DH, DH)] * bc

        pltpu.sync_copy(xw_v, agg_sh.at[dst_v.at[j]], add=True)

    plsc.subcore_barrier()

    @pl.loop(0, RP // 64)
    def _(p):
        pltpu.sync_copy(agg_sh.at[pl.ds(base + p * 64, 64)], z_v)
        pltpu.sync_copy(z_v, agg_hbm.at[c, pl.ds(base + p * 64, 64)])


def _scB(xw, ex16, rcp16, src_ab, dst_ab):
    k = pl.kernel(
        _scB_body,
        out_type=_f32((NC, NP, D)),
        mesh=_mesh,
        scratch_types=[
            pltpu.VMEM((NCH_AB, CH), jnp.int32),
            pltpu.VMEM((NCH_AB, CH), jnp.int32),
            pltpu.VMEM((CH, D), jnp.float32),
            pltpu.VMEM((CH, DH), jnp.float32),
            pltpu.VMEM((CH, DH), jnp.float32),
            pltpu.VMEM((CH, DH), jnp.float32),
            pltpu.VMEM((64, D), jnp.float32),
            pltpu.VMEM_SHARED((NP, D), jnp.float32),
        ],
        compiler_params=_sc_params(),
    )
    return k(xw, ex16, rcp16, src_ab, dst_ab)


def _scC_body(st_hbm, inv_hbm, dw_hbm, pos_hbm, src_hbm, dst_hbm, att_hbm,
              src_v, dst_v, inv_v, cur_v, bias_v, nb_v, z_v, g_v, dw_v,
              cur_sh, nxt_sh):
    c = lax.axis_index("c")
    s = lax.axis_index("s")
    base = s * RP
    zero = jnp.zeros((DH,), jnp.float32)
    lane = lax.iota(jnp.int32, DH)
    m8 = lane == H

    pltpu.sync_copy(src_hbm.at[s], src_v)
    pltpu.sync_copy(dst_hbm.at[s], dst_v)
    pltpu.sync_copy(inv_hbm.at[pl.ds(base, RP)], inv_v)
    pltpu.sync_copy(st_hbm.at[pl.ds(base, RP)], cur_v)
    pltpu.sync_copy(cur_v, cur_sh.at[pl.ds(base, RP)])
    pltpu.sync_copy(dw_hbm, dw_v)

    @pl.loop(0, RP)
    def _(r):
        z_v[r] = zero
        bias_v[r] = zero
    pltpu.sync_copy(z_v, nxt_sh.at[pl.ds(base, RP)])
    plsc.subcore_barrier()

    @pl.loop(0, NUM_HOPS)
    def _(hop):
        @pl.loop(0, NCH_C)
        def _(j):
            pltpu.sync_copy(cur_sh.at[src_v.at[j]], g_v)
            pltpu.sync_copy(g_v, nxt_sh.at[dst_v.at[j]], add=True)

        plsc.subcore_barrier()
        pltpu.sync_copy(nxt_sh.at[pl.ds(base, RP)], nb_v)
        dwrow = dw_v[hop]

        @pl.loop(0, RP)
        def _(r):
            srow = nb_v[r]
            crow = cur_v[r]
            fixed = jnp.where(m8, jnp.minimum(crow + srow, 1.0),
                              srow * inv_v[r])
            bias_v[r] = bias_v[r] + dwrow * fixed
            cur_v[r] = fixed
            nb_v[r] = fixed

        pltpu.sync_copy(nb_v, cur_sh.at[pl.ds(base, RP)])
        pltpu.sync_copy(z_v, nxt_sh.at[pl.ds(base, RP)])
        plsc.subcore_barrier()

    @pl.when(c == 0)
    def _():
        pltpu.sync_copy(pos_hbm.at[pl.ds(base, RP)], z_v)

        @pl.loop(0, RP)
        def _(r):
            nb_v[r] = bias_v[r] + z_v[r] + jnp.where(m8, cur_v[r], 0.0)

        pltpu.sync_copy(nb_v, att_hbm.at[pl.ds(base, RP)])


def _scC(state0, inv16, dw16, pos16, src_c, dst_c):
    k = pl.kernel(
        _scC_body,
        out_type=_f32((NP, DH)),
        mesh=_mesh,
        scratch_types=[
            pltpu.VMEM((NCH_C, CH), jnp.int32),
            pltpu.VMEM((NCH_C, CH), jnp.int32),
            pltpu.VMEM((RP, DH), jnp.float32),
            pltpu.VMEM((RP, DH), jnp.float32),
            pltpu.VMEM((RP, DH), jnp.float32),
            pltpu.VMEM((RP, DH), jnp.float32),
            pltpu.VMEM((RP, DH), jnp.float32),
            pltpu.VMEM((CH, DH), jnp.float32),
            pltpu.VMEM((NUM_HOPS, DH), jnp.float32),
            pltpu.VMEM_SHARED((NP, DH), jnp.float32),
            pltpu.VMEM_SHARED((NP, DH), jnp.float32),
        ],
        compiler_params=_sc_params(),
    )
    return k(state0, inv16, dw16, pos16, src_c, dst_c)


@jax.jit
def kernel(x, edge_index, position_bias, W_gat, a_src, a_dst, W_fuse, b_fuse,
           diffusion_weight):
    f32 = jnp.float32
    G = jnp.repeat(jnp.eye(H, dtype=f32), DH, axis=0)
    a_s16 = jnp.pad(G * a_src.reshape(-1)[:, None], ((0, 0), (0, H)))
    a_d16 = jnp.pad(G * a_dst.reshape(-1)[:, None], ((0, 0), (0, H)))
    m16 = jnp.pad(G / DH, ((0, 0), (0, H)))
    dw16 = jnp.pad(diffusion_weight.astype(f32), ((0, 0), (0, H)))
    pos16 = jnp.pad(position_bias.astype(f32), ((0, NP - N), (0, H)))
    src_ab = edge_index[0].reshape(NC, NS, NCH_AB, CH)
    dst_ab = edge_index[1].reshape(NC, NS, NCH_AB, CH)
    src_c = edge_index[0].reshape(NS, NCH_C, CH)
    dst_c = edge_index[1].reshape(NS, NCH_C, CH)

    xw, as16, ad16 = _tc1(x, W_gat, a_s16, a_d16)
    as16 = jnp.pad(as16, ((0, NP - N), (0, 0)))
    ad16 = jnp.pad(ad16, ((0, NP - N), (0, 0)))
    ex16, dnp = _scA(as16, ad16, src_ab, dst_ab)
    rcp16, inv16 = _scA2(dnp)
    aggp = _scB(xw, ex16, rcp16, src_ab, dst_ab)
    state0 = _tc2(x, aggp[0, :N], aggp[1, :N], W_fuse[:D], W_fuse[D:],
                  b_fuse.reshape(1, D), m16)
    state0 = jnp.pad(state0, ((0, NP - N), (0, 0)))
    att16 = _scC(state0, inv16, dw16, pos16, src_c, dst_c)
    return (att16[:N, :H], att16[:N, H])


# speedup vs baseline: 366.6996x; 1.9109x over previous
import dataclasses

import jax
import jax.numpy as jnp
from jax import lax
from jax.experimental import pallas as pl
from jax.experimental.pallas import tpu as pltpu
from jax.experimental.pallas import tpu_sc as plsc

H = 8
DH = 16
NUM_HOPS = 8
N = 10000
E = 320000
D = 128
DHALF = 64

NC = 2
NS = 16
NW = NC * NS
NP = 10240
RP = NP // NS
RP2 = NP // NW
CH = 100
NCH_A = E // (NW * CH)
NCH_B = E // (NS * CH)

_mesh = plsc.VectorSubcoreMesh(core_axis_name="c", subcore_axis_name="s")


def _sc_params():
    cp = pltpu.CompilerParams()
    fields = pltpu.CompilerParams.__dataclass_fields__
    if "needs_layout_passes" in fields:
        cp = dataclasses.replace(cp, needs_layout_passes=False)
    if "use_tc_tiling_on_sc" in fields:
        cp = dataclasses.replace(cp, use_tc_tiling_on_sc=False)
    return cp


def _f32(shape):
    return jax.ShapeDtypeStruct(shape, jnp.float32)


def _tc1_body(x_ref, w_ref, as_ref, ad_ref, xw_ref, as16_ref, ad16_ref):
    xw = jnp.dot(x_ref[...], w_ref[...], preferred_element_type=jnp.float32)
    xw_ref[...] = xw
    as16_ref[...] = jnp.dot(xw, as_ref[...], preferred_element_type=jnp.float32)
    ad16_ref[...] = jnp.dot(xw, ad_ref[...], preferred_element_type=jnp.float32)


def _tc1(x, w_gat, a_s16, a_d16):
    blk = 1000
    return pl.pallas_call(
        _tc1_body,
        grid=(N // blk,),
        in_specs=[
            pl.BlockSpec((blk, D), lambda i: (i, 0)),
            pl.BlockSpec((D, D), lambda i: (0, 0)),
            pl.BlockSpec((D, DH), lambda i: (0, 0)),
            pl.BlockSpec((D, DH), lambda i: (0, 0)),
        ],
        out_specs=[
            pl.BlockSpec((blk, D), lambda i: (i, 0)),
            pl.BlockSpec((blk, DH), lambda i: (i, 0)),
            pl.BlockSpec((blk, DH), lambda i: (i, 0)),
        ],
        out_shape=[_f32((N, D)), _f32((N, DH)), _f32((N, DH))],
    )(x, w_gat, a_s16, a_d16)


def _tc2_body(x_ref, a0_ref, a1_ref, rcp_ref, rlo_ref, rhi_ref,
              wt_ref, wblo_ref, wbhi_ref, b_ref, m16_ref, st_ref):
    rlo = jnp.dot(rcp_ref[...], rlo_ref[...], preferred_element_type=jnp.float32)
    rhi = jnp.dot(rcp_ref[...], rhi_ref[...], preferred_element_type=jnp.float32)
    f = (jnp.dot(x_ref[...], wt_ref[...], preferred_element_type=jnp.float32)
         + jnp.dot(a0_ref[...] * rlo, wblo_ref[...],
                   preferred_element_type=jnp.float32)
         + jnp.dot(a1_ref[...] * rhi, wbhi_ref[...],
                   preferred_element_type=jnp.float32)
         + b_ref[...])
    f = jnp.maximum(f, 0.0)
    st = jnp.dot(f, m16_ref[...], preferred_element_type=jnp.float32)
    row = lax.broadcasted_iota(jnp.int32, st.shape, 0)
    lane = lax.broadcasted_iota(jnp.int32, st.shape, 1)
    first = (pl.program_id(0) == 0) & (row == 0) & (lane == H)
    st_ref[...] = jnp.where(first, 1.0, st)


def _tc2(x, a0, a1, rcp, r_lo, r_hi, wf_top, wb_lo, wb_hi, b2d, m16):
    blk = 1000
    return pl.pallas_call(
        _tc2_body,
        grid=(N // blk,),
        in_specs=[
            pl.BlockSpec((blk, D), lambda i: (i, 0)),
            pl.BlockSpec((blk, DHALF), lambda i: (i, 0)),
            pl.BlockSpec((blk, DHALF), lambda i: (i, 0)),
            pl.BlockSpec((blk, DH), lambda i: (i, 0)),
            pl.BlockSpec((DH, DHALF), lambda i: (0, 0)),
            pl.BlockSpec((DH, DHALF), lambda i: (0, 0)),
            pl.BlockSpec((D, D), lambda i: (0, 0)),
            pl.BlockSpec((DHALF, D), lambda i: (0, 0)),
            pl.BlockSpec((DHALF, D), lambda i: (0, 0)),
            pl.BlockSpec((1, D), lambda i: (0, 0)),
            pl.BlockSpec((D, DH), lambda i: (0, 0)),
        ],
        out_specs=pl.BlockSpec((blk, DH), lambda i: (i, 0)),
        out_shape=_f32((N, DH)),
    )(x, a0, a1, rcp, r_lo, r_hi, wf_top, wb_lo, wb_hi, b2d, m16)


def _scA_body(as_hbm, ad_hbm, src_hbm, dst_hbm, ex_hbm, dnp_hbm,
              src_v, dst_v, ga0, ga1, gb0, gb1, ac0, ac1, t_v, acc_sh,
              gsa0, gsa1, gsb0, gsb1, sse0, sse1, ssc0, ssc1):
    c = lax.axis_index("c")
    s = lax.axis_index("s")
    base = s * RP
    zero = jnp.zeros((DH,), jnp.float32)
    ga = (ga0, ga1)
    gb = (gb0, gb1)
    ac = (ac0, ac1)
    gsa = (gsa0, gsa1)
    gsb = (gsb0, gsb1)
    sse = (sse0, sse1)
    ssc = (ssc0, ssc1)

    def g_start(b, jj):
        pltpu.async_copy(as_hbm.at[src_v.at[jj]], ga[b], gsa[b])
        pltpu.async_copy(ad_hbm.at[dst_v.at[jj]], gb[b], gsb[b])

    def g_wait(b, jj):
        pltpu.make_async_copy(as_hbm.at[src_v.at[jj]], ga[b], gsa[b]).wait()
        pltpu.make_async_copy(ad_hbm.at[dst_v.at[jj]], gb[b], gsb[b]).wait()

    def s_start(b, jj):
        pltpu.async_copy(ac[b], ex_hbm.at[c, s, jj], sse[b])
        pltpu.async_copy(ac[b], acc_sh.at[dst_v.at[jj]], ssc[b], add=True)

    def s_wait(b, jj):
        pltpu.make_async_copy(ac[b], ex_hbm.at[c, s, jj], sse[b]).wait()
        pltpu.make_async_copy(ac[b], acc_sh.at[dst_v.at[jj]], ssc[b]).wait()

    def compute(b):
        @pl.loop(0, CH)
        def _(i):
            v = ga[b][i] + gb[b][i]
            v = jnp.where(v > 0.0, v, 0.2 * v)
            ac[b][i] = jnp.exp(v)

    @pl.loop(0, RP)
    def _(r):
        t_v[r] = zero
    pltpu.sync_copy(t_v, acc_sh.at[pl.ds(base, RP)])
    pltpu.sync_copy(src_hbm.at[c, s], src_v)
    pltpu.sync_copy(dst_hbm.at[c, s], dst_v)
    plsc.subcore_barrier()

    g_start(0, 0)
    g_start(1, 1)
    for b in (0, 1):
        g_wait(b, b)
        compute(b)
        s_start(b, b)
        g_start(b, b + 2)

    @pl.loop(1, NCH_A // 2)
    def _(p):
        for b in (0, 1):
            jj = 2 * p + b
            g_wait(b, jj)
            s_wait(b, jj - 2)
            compute(b)
            s_start(b, jj)

            @pl.when(jj + 2 < NCH_A)
            def _(b=b, jj=jj):
                g_start(b, jj + 2)

    s_wait(0, NCH_A - 2)
    s_wait(1, NCH_A - 1)
    plsc.subcore_barrier()
    pltpu.sync_copy(acc_sh.at[pl.ds(base, RP)], t_v)
    pltpu.sync_copy(t_v, dnp_hbm.at[c, pl.ds(base, RP)])


def _scA(as16, ad16, src_a, dst_a):
    k = pl.kernel(
        _scA_body,
        out_type=(_f32((NC, NS, NCH_A, CH, DH)), _f32((NC, NP, DH))),
        mesh=_mesh,
        scratch_types=[
            pltpu.VMEM((NCH_A, CH), jnp.int32),
            pltpu.VMEM((NCH_A, CH), jnp.int32),
            pltpu.VMEM((CH, DH), jnp.float32),
            pltpu.VMEM((CH, DH), jnp.float32),
            pltpu.VMEM((CH, DH), jnp.float32),
            pltpu.VMEM((CH, DH), jnp.float32),
            pltpu.VMEM((CH, DH), jnp.float32),
            pltpu.VMEM((CH, DH), jnp.float32),
            pltpu.VMEM((RP, DH), jnp.float32),
            pltpu.VMEM_SHARED((NP, DH), jnp.float32),
            pltpu.SemaphoreType.DMA,
            pltpu.SemaphoreType.DMA,
            pltpu.SemaphoreType.DMA,
            pltpu.SemaphoreType.DMA,
            pltpu.SemaphoreType.DMA,
            pltpu.SemaphoreType.DMA,
            pltpu.SemaphoreType.DMA,
            pltpu.SemaphoreType.DMA,
        ],
        compiler_params=_sc_params(),
    )
    return k(as16, ad16, src_a, dst_a)


def _scA2_body(dnp_hbm, rcp_hbm, inv_hbm, d0_v, d1_v, rcp_v, inv_v):
    c = lax.axis_index("c")
    s = lax.axis_index("s")
    base = (s * NC + c) * RP2
    pltpu.sync_copy(dnp_hbm.at[0, pl.ds(base, RP2)], d0_v)
    pltpu.sync_copy(dnp_hbm.at[1, pl.ds(base, RP2)], d1_v)
    lane = lax.iota(jnp.int32, DH)
    is_den = lane < H

    @pl.loop(0, RP2)
    def _(r):
        drow = d0_v[r] + d1_v[r]
        d0_v[r] = drow
        degv = plsc.load_gather(
            d0_v, [jnp.full((DH,), r, jnp.int32), jnp.full((DH,), H, jnp.int32)])
        degc = jnp.maximum(degv, 1.0)
        rcp_v[r] = 1.0 / jnp.where(is_den, drow + 1e-16, degc)
        inv_v[r] = 1.0 / degc

    pltpu.sync_copy(rcp_v, rcp_hbm.at[pl.ds(base, RP2)])
    pltpu.sync_copy(inv_v, inv_hbm.at[pl.ds(base, RP2)])


def _scA2(dnp):
    k = pl.kernel(
        _scA2_body,
        out_type=(_f32((NP, DH)), _f32((NP, DH))),
        mesh=_mesh,
        scratch_types=[
            pltpu.VMEM((RP2, DH), jnp.float32),
            pltpu.VMEM((RP2, DH), jnp.float32),
            pltpu.VMEM((RP2, DH), jnp.float32),
            pltpu.VMEM((RP2, DH), jnp.float32),
        ],
        compiler_params=_sc_params(),
    )
    return k(dnp)


def _scB_body(xws_hbm, ex_hbm, src_hbm, dst_hbm, agg_hbm,
              src_v, dst_v, gx0, gx1, ob0, ob1, eb0, eb1, cob, agg_sh,
              gsx0, gsx1, gse0, gse1, ssc0, ssc1):
    c = lax.axis_index("c")
    s = lax.axis_index("s")
    base = s * RP
    zero = jnp.zeros((DH,), jnp.float32)
    gx = (gx0, gx1)
    ob = (ob0, ob1)
    eb = (eb0, eb1)
    gsx = (gsx0, gsx1)
    gse = (gse0, gse1)
    ssc = (ssc0, ssc1)

    def g_start(b, jj):
        pltpu.async_copy(xws_hbm.at[src_v.at[jj]], gx[b], gsx[b])
        pltpu.async_copy(ex_hbm.at[s, jj], eb[b], gse[b])

    def g_wait(b, jj):
        pltpu.make_async_copy(xws_hbm.at[src_v.at[jj]], gx[b], gsx[b]).wait()
        pltpu.make_async_copy(ex_hbm.at[s, jj], eb[b], gse[b]).wait()

    def s_start(b, jj):
        pltpu.async_copy(ob[b], agg_sh.at[dst_v.at[jj]], ssc[b], add=True)

    def s_wait(b, jj):
        pltpu.make_async_copy(ob[b], agg_sh.at[dst_v.at[jj]], ssc[b]).wait()

    def compute(b):
        hbase = c * 4

        @pl.loop(0, CH)
        def _(e):
            efull = jnp.full((DH,), e, jnp.int32)
            for h in range(4):
                bc = plsc.load_gather(
                    eb[b], [efull, jnp.full((DH,), h, jnp.int32) + hbase])
                ob[b][e, pl.ds(h * DH, DH)] = gx[b][e, pl.ds(h * DH, DH)] * bc

    @pl.loop(0, 64)
    def _(r):
        for q in range(DHALF // DH):
            cob[r, pl.ds(q * DH, DH)] = zero

    @pl.loop(0, RP // 64)
    def _(p):
        pltpu.sync_copy(cob, agg_sh.at[pl.ds(base + p * 64, 64)])
    pltpu.sync_copy(src_hbm.at[c, s], src_v)
    pltpu.sync_copy(dst_hbm.at[s], dst_v)
    plsc.subcore_barrier()

    g_start(0, 0)
    g_start(1, 1)
    for b in (0, 1):
        g_wait(b, b)
        compute(b)
        s_start(b, b)
        g_start(b, b + 2)

    @pl.loop(1, NCH_B // 2)
    def _(p):
        for b in (0, 1):
            jj = 2 * p + b
            g_wait(b, jj)
            s_wait(b, jj - 2)
            compute(b)
            s_start(b, jj)

            @pl.when(jj + 2 < NCH_B)
            def _(b=b, jj=jj):
                g_start(b, jj + 2)

    s_wait(0, NCH_B - 2)
    s_wait(1, NCH_B - 1)
    plsc.subcore_barrier()

    @pl.loop(0, RP // 64)
    def _(p):
        pltpu.sync_copy(agg_sh.at[pl.ds(base + p * 64, 64)], cob)
        pltpu.sync_copy(cob, agg_hbm.at[c, pl.ds(base + p * 64, 64)])


def _scB(xws, exB, srcB, dstB):
    k = pl.kernel(
        _scB_body,
        out_type=_f32((NC, NP, DHALF)),
        mesh=_mesh,
        scratch_types=[
            pltpu.VMEM((NCH_B, CH), jnp.int32),
            pltpu.VMEM((NCH_B, CH), jnp.int32),
            pltpu.VMEM((CH, DHALF), jnp.float32),
            pltpu.VMEM((CH, DHALF), jnp.float32),
            pltpu.VMEM((CH, DHALF), jnp.float32),
            pltpu.VMEM((CH, DHALF), jnp.float32),
            pltpu.VMEM((CH, DH), jnp.float32),
            pltpu.VMEM((CH, DH), jnp.float32),
            pltpu.VMEM((64, DHALF), jnp.float32),
            pltpu.VMEM_SHARED((NP, DHALF), jnp.float32),
            pltpu.SemaphoreType.DMA,
            pltpu.SemaphoreType.DMA,
            pltpu.SemaphoreType.DMA,
            pltpu.SemaphoreType.DMA,
            pltpu.SemaphoreType.DMA,
            pltpu.SemaphoreType.DMA,
        ],
        compiler_params=_sc_params(),
    )
    return k(xws, exB, srcB, dstB)


def _scC_body(st_hbm, inv_hbm, dw_hbm, pos_hbm, src_hbm, dst_hbm, att_hbm,
              src_v, dst_v, inv_v, cur_v, bias_v, nb_v, z_v, dw_v,
              g0, g1, g2, g3, cur_sh, nxt_sh,
              gs0, gs1, gs2, gs3, ss0, ss1, ss2, ss3):
    c = lax.axis_index("c")
    s = lax.axis_index("s")
    base = s * RP
    zero = jnp.zeros((DH,), jnp.float32)
    lane = lax.iota(jnp.int32, DH)
    m8 = lane == H
    g = (g0, g1, g2, g3)
    gs = (gs0, gs1, gs2, gs3)
    ss = (ss0, ss1, ss2, ss3)

    def g_start(b, jj):
        pltpu.async_copy(cur_sh.at[src_v.at[jj]], g[b], gs[b])

    def g_wait(b, jj):
        pltpu.make_async_copy(cur_sh.at[src_v.at[jj]], g[b], gs[b]).wait()

    def s_start(b, jj):
        pltpu.async_copy(g[b], nxt_sh.at[dst_v.at[jj]], ss[b], add=True)

    def s_wait(b, jj):
        pltpu.make_async_copy(g[b], nxt_sh.at[dst_v.at[jj]], ss[b]).wait()

    pltpu.sync_copy(src_hbm.at[s], src_v)
    pltpu.sync_copy(dst_hbm.at[s], dst_v)
    pltpu.sync_copy(inv_hbm.at[pl.ds(base, RP)], inv_v)
    pltpu.sync_copy(st_hbm.at[pl.ds(base, RP)], cur_v)
    pltpu.sync_copy(cur_v, cur_sh.at[pl.ds(base, RP)])
    pltpu.sync_copy(dw_hbm, dw_v)

    @pl.loop(0, RP)
    def _(r):
        z_v[r] = zero
        bias_v[r] = zero
    pltpu.sync_copy(z_v, nxt_sh.at[pl.ds(base, RP)])
    plsc.subcore_barrier()

    @pl.loop(0, NUM_HOPS)
    def _(hop):
        g_start(0, 0)
        g_start(1, 1)
        g_wait(0, 0)
        s_start(0, 0)
        g_start(2, 2)
        g_wait(1, 1)
        s_start(1, 1)
        g_start(3, 3)
        g_wait(2, 2)
        s_start(2, 2)
        s_wait(0, 0)
        g_start(0, 4)
        g_wait(3, 3)
        s_start(3, 3)
        s_wait(1, 1)
        g_start(1, 5)

        @pl.loop(1, NCH_B // 4)
        def _(p):
            for b in range(4):
                jj = 4 * p + b
                g_wait(b, jj)
                s_start(b, jj)
                b2 = (b + 2) % 4
                s_wait(b2, jj - 2)

                @pl.when(jj + 2 < NCH_B)
                def _(b2=b2, jj=jj):
                    g_start(b2, jj + 2)

        s_wait(2, NCH_B - 2)
        s_wait(3, NCH_B - 1)
        plsc.subcore_barrier()
        pltpu.sync_copy(nxt_sh.at[pl.ds(base, RP)], nb_v)
        dwrow = dw_v[hop]

        @pl.loop(0, RP)
        def _(r):
            srow = nb_v[r]
            crow = cur_v[r]
            fixed = jnp.where(m8, jnp.minimum(crow + srow, 1.0),
                              srow * inv_v[r])
            bias_v[r] = bias_v[r] + dwrow * fixed
            cur_v[r] = fixed
            nb_v[r] = fixed

        pltpu.sync_copy(nb_v, cur_sh.at[pl.ds(base, RP)])
        pltpu.sync_copy(z_v, nxt_sh.at[pl.ds(base, RP)])
        plsc.subcore_barrier()

    @pl.when(c == 0)
    def _():
        pltpu.sync_copy(pos_hbm.at[pl.ds(base, RP)], z_v)

        @pl.loop(0, RP)
        def _(r):
            nb_v[r] = bias_v[r] + z_v[r] + jnp.where(m8, cur_v[r], 0.0)

        pltpu.sync_copy(nb_v, att_hbm.at[pl.ds(base, RP)])


def _scC(state0, inv16, dw16, pos16, src_c, dst_c):
    k = pl.kernel(
        _scC_body,
        out_type=_f32((NP, DH)),
        mesh=_mesh,
        scratch_types=[
            pltpu.VMEM((NCH_B, CH), jnp.int32),
            pltpu.VMEM((NCH_B, CH), jnp.int32),
            pltpu.VMEM((RP, DH), jnp.float32),
            pltpu.VMEM((RP, DH), jnp.float32),
            pltpu.VMEM((RP, DH), jnp.float32),
            pltpu.VMEM((RP, DH), jnp.float32),
            pltpu.VMEM((RP, DH), jnp.float32),
            pltpu.VMEM((NUM_HOPS, DH), jnp.float32),
            pltpu.VMEM((CH, DH), jnp.float32),
            pltpu.VMEM((CH, DH), jnp.float32),
            pltpu.VMEM((CH, DH), jnp.float32),
            pltpu.VMEM((CH, DH), jnp.float32),
            pltpu.VMEM_SHARED((NP, DH), jnp.float32),
            pltpu.VMEM_SHARED((NP, DH), jnp.float32),
            pltpu.SemaphoreType.DMA,
            pltpu.SemaphoreType.DMA,
            pltpu.SemaphoreType.DMA,
            pltpu.SemaphoreType.DMA,
            pltpu.SemaphoreType.DMA,
            pltpu.SemaphoreType.DMA,
            pltpu.SemaphoreType.DMA,
            pltpu.SemaphoreType.DMA,
        ],
        compiler_params=_sc_params(),
    )
    return k(state0, inv16, dw16, pos16, src_c, dst_c)


@jax.jit
def kernel(x, edge_index, position_bias, W_gat, a_src, a_dst, W_fuse, b_fuse,
           diffusion_weight):
    f32 = jnp.float32
    G = jnp.repeat(jnp.eye(H, dtype=f32), DH, axis=0)
    a_s16 = jnp.pad(G * a_src.reshape(-1)[:, None], ((0, 0), (0, H)))
    a_d16 = jnp.pad(G * a_dst.reshape(-1)[:, None], ((0, 0), (0, H)))
    m16 = jnp.pad(G / DH, ((0, 0), (0, H)))
    dw16 = jnp.pad(diffusion_weight.astype(f32), ((0, 0), (0, H)))
    pos16 = jnp.pad(position_bias.astype(f32), ((0, NP - N), (0, H)))
    g8 = jnp.repeat(jnp.eye(H, dtype=f32), DH, axis=1)
    r_lo = jnp.pad(g8[:, :DHALF], ((0, H), (0, 0)))
    r_hi = jnp.pad(g8[:, DHALF:], ((0, H), (0, 0)))
    src_a = edge_index[0].reshape(NC, NS, NCH_A, CH)
    dst_a = edge_index[1].reshape(NC, NS, NCH_A, CH)
    src_t = edge_index[0].reshape(NS, NCH_B, CH)
    dst_t = edge_index[1].reshape(NS, NCH_B, CH)
    src_b = jnp.stack([src_t, src_t + N])

    xw, as16, ad16 = _tc1(x, W_gat, a_s16, a_d16)
    as16 = jnp.pad(as16, ((0, NP - N), (0, 0)))
    ad16 = jnp.pad(ad16, ((0, NP - N), (0, 0)))
    xws = jnp.concatenate([xw[:, :DHALF], xw[:, DHALF:]], axis=0)
    ex16, dnp = _scA(as16, ad16, src_a, dst_a)
    rcp16, inv16 = _scA2(dnp)
    exB = ex16.reshape(NS, NCH_B, CH, DH)
    aggp = _scB(xws, exB, src_b, dst_t)
    state0 = _tc2(x, aggp[0, :N], aggp[1, :N], rcp16[:N], r_lo, r_hi,
                  W_fuse[:D], W_fuse[D:D + DHALF], W_fuse[D + DHALF:],
                  b_fuse.reshape(1, D), m16)
    state0 = jnp.pad(state0, ((0, NP - N), (0, 0)))
    att16 = _scC(state0, inv16, dw16, pos16, src_t, dst_t)
    return (att16[:N, :H], att16[:N, H])


# speedup vs baseline: 522.8780x; 1.4259x over previous
import dataclasses

import jax
import jax.numpy as jnp
from jax import lax
from jax.experimental import pallas as pl
from jax.experimental.pallas import tpu as pltpu
from jax.experimental.pallas import tpu_sc as plsc

H = 8
DH = 16
NUM_HOPS = 8
N = 10000
E = 320000
D = 128
DHALF = 64

NC = 2
NS = 16
NW = NC * NS
NP = 10240
RP = NP // NS
RP2 = NP // NW
CH = 125
NCH_A = E // (NW * CH)
NCH_B = E // (NS * CH)

_mesh = plsc.VectorSubcoreMesh(core_axis_name="c", subcore_axis_name="s")


def _sc_params():
    cp = pltpu.CompilerParams()
    fields = pltpu.CompilerParams.__dataclass_fields__
    if "needs_layout_passes" in fields:
        cp = dataclasses.replace(cp, needs_layout_passes=False)
    if "use_tc_tiling_on_sc" in fields:
        cp = dataclasses.replace(cp, use_tc_tiling_on_sc=False)
    return cp


def _f32(shape):
    return jax.ShapeDtypeStruct(shape, jnp.float32)


def _tc1_body(x_ref, w_ref, as_ref, ad_ref, xw_ref, as16_ref, ad16_ref):
    xw = jnp.dot(x_ref[...], w_ref[...], preferred_element_type=jnp.float32)
    xw_ref[...] = xw
    as16_ref[...] = jnp.dot(xw, as_ref[...], preferred_element_type=jnp.float32)
    ad16_ref[...] = jnp.dot(xw, ad_ref[...], preferred_element_type=jnp.float32)


def _tc1(x, w_gat, a_s16, a_d16):
    blk = 1000
    return pl.pallas_call(
        _tc1_body,
        grid=(N // blk,),
        in_specs=[
            pl.BlockSpec((blk, D), lambda i: (i, 0)),
            pl.BlockSpec((D, D), lambda i: (0, 0)),
            pl.BlockSpec((D, DH), lambda i: (0, 0)),
            pl.BlockSpec((D, DH), lambda i: (0, 0)),
        ],
        out_specs=[
            pl.BlockSpec((blk, D), lambda i: (i, 0)),
            pl.BlockSpec((blk, DH), lambda i: (i, 0)),
            pl.BlockSpec((blk, DH), lambda i: (i, 0)),
        ],
        out_shape=[_f32((N, D)), _f32((N, DH)), _f32((N, DH))],
    )(x, w_gat, a_s16, a_d16)


def _tc2_body(x_ref, a0_ref, a1_ref, rcp_ref, rlo_ref, rhi_ref,
              wt_ref, wblo_ref, wbhi_ref, b_ref, m16_ref, st_ref):
    rlo = jnp.dot(rcp_ref[...], rlo_ref[...], preferred_element_type=jnp.float32)
    rhi = jnp.dot(rcp_ref[...], rhi_ref[...], preferred_element_type=jnp.float32)
    f = (jnp.dot(x_ref[...], wt_ref[...], preferred_element_type=jnp.float32)
         + jnp.dot(a0_ref[...] * rlo, wblo_ref[...],
                   preferred_element_type=jnp.float32)
         + jnp.dot(a1_ref[...] * rhi, wbhi_ref[...],
                   preferred_element_type=jnp.float32)
         + b_ref[...])
    f = jnp.maximum(f, 0.0)
    st = jnp.dot(f, m16_ref[...], preferred_element_type=jnp.float32)
    row = lax.broadcasted_iota(jnp.int32, st.shape, 0)
    lane = lax.broadcasted_iota(jnp.int32, st.shape, 1)
    first = (pl.program_id(0) == 0) & (row == 0) & (lane == H)
    st_ref[...] = jnp.where(first, 1.0, st)


def _tc2(x, a0, a1, rcp, r_lo, r_hi, wf_top, wb_lo, wb_hi, b2d, m16):
    blk = 1000
    return pl.pallas_call(
        _tc2_body,
        grid=(N // blk,),
        in_specs=[
            pl.BlockSpec((blk, D), lambda i: (i, 0)),
            pl.BlockSpec((blk, DHALF), lambda i: (i, 0)),
            pl.BlockSpec((blk, DHALF), lambda i: (i, 0)),
            pl.BlockSpec((blk, DH), lambda i: (i, 0)),
            pl.BlockSpec((DH, DHALF), lambda i: (0, 0)),
            pl.BlockSpec((DH, DHALF), lambda i: (0, 0)),
            pl.BlockSpec((D, D), lambda i: (0, 0)),
            pl.BlockSpec((DHALF, D), lambda i: (0, 0)),
            pl.BlockSpec((DHALF, D), lambda i: (0, 0)),
            pl.BlockSpec((1, D), lambda i: (0, 0)),
            pl.BlockSpec((D, DH), lambda i: (0, 0)),
        ],
        out_specs=pl.BlockSpec((blk, DH), lambda i: (i, 0)),
        out_shape=_f32((N, DH)),
    )(x, a0, a1, rcp, r_lo, r_hi, wf_top, wb_lo, wb_hi, b2d, m16)


def _scA_body(as_hbm, ad_hbm, src_hbm, dst_hbm, ex_hbm, dnp_hbm,
              src_v, dst_v, ga0, ga1, gb0, gb1, ac0, ac1, t_v, acc_sh,
              gsa0, gsa1, gsb0, gsb1, sse0, sse1, ssc0, ssc1):
    c = lax.axis_index("c")
    s = lax.axis_index("s")
    base = s * RP
    zero = jnp.zeros((DH,), jnp.float32)
    ga = (ga0, ga1)
    gb = (gb0, gb1)
    ac = (ac0, ac1)
    gsa = (gsa0, gsa1)
    gsb = (gsb0, gsb1)
    sse = (sse0, sse1)
    ssc = (ssc0, ssc1)

    def g_start(b, jj):
        pltpu.async_copy(as_hbm.at[src_v.at[jj]], ga[b], gsa[b])
        pltpu.async_copy(ad_hbm.at[dst_v.at[jj]], gb[b], gsb[b])

    def g_wait(b, jj):
        pltpu.make_async_copy(as_hbm.at[src_v.at[jj]], ga[b], gsa[b]).wait()
        pltpu.make_async_copy(ad_hbm.at[dst_v.at[jj]], gb[b], gsb[b]).wait()

    def s_start(b, jj):
        pltpu.async_copy(ac[b], ex_hbm.at[c, s, jj], sse[b])
        pltpu.async_copy(ac[b], acc_sh.at[dst_v.at[jj]], ssc[b], add=True)

    def s_wait(b, jj):
        pltpu.make_async_copy(ac[b], ex_hbm.at[c, s, jj], sse[b]).wait()
        pltpu.make_async_copy(ac[b], acc_sh.at[dst_v.at[jj]], ssc[b]).wait()

    def compute(b):
        @pl.loop(0, CH)
        def _(i):
            v = ga[b][i] + gb[b][i]
            v = jnp.where(v > 0.0, v, 0.2 * v)
            ac[b][i] = jnp.exp(v)

    @pl.loop(0, RP)
    def _(r):
        t_v[r] = zero
    pltpu.sync_copy(t_v, acc_sh.at[pl.ds(base, RP)])
    pltpu.sync_copy(src_hbm.at[c, s], src_v)
    pltpu.sync_copy(dst_hbm.at[c, s], dst_v)
    plsc.subcore_barrier()

    g_start(0, 0)
    g_start(1, 1)
    for b in (0, 1):
        g_wait(b, b)
        compute(b)
        s_start(b, b)
        g_start(b, b + 2)

    @pl.loop(1, NCH_A // 2)
    def _(p):
        for b in (0, 1):
            jj = 2 * p + b
            g_wait(b, jj)
            s_wait(b, jj - 2)
            compute(b)
            s_start(b, jj)

            @pl.when(jj + 2 < NCH_A)
            def _(b=b, jj=jj):
                g_start(b, jj + 2)

    s_wait(0, NCH_A - 2)
    s_wait(1, NCH_A - 1)
    plsc.subcore_barrier()
    pltpu.sync_copy(acc_sh.at[pl.ds(base, RP)], t_v)
    pltpu.sync_copy(t_v, dnp_hbm.at[c, pl.ds(base, RP)])


def _scA(as16, ad16, src_a, dst_a):
    k = pl.kernel(
        _scA_body,
        out_type=(_f32((NC, NS, NCH_A, CH, DH)), _f32((NC, NP, DH))),
        mesh=_mesh,
        scratch_types=[
            pltpu.VMEM((NCH_A, CH), jnp.int32),
            pltpu.VMEM((NCH_A, CH), jnp.int32),
            pltpu.VMEM((CH, DH), jnp.float32),
            pltpu.VMEM((CH, DH), jnp.float32),
            pltpu.VMEM((CH, DH), jnp.float32),
            pltpu.VMEM((CH, DH), jnp.float32),
            pltpu.VMEM((CH, DH), jnp.float32),
            pltpu.VMEM((CH, DH), jnp.float32),
            pltpu.VMEM((RP, DH), jnp.float32),
            pltpu.VMEM_SHARED((NP, DH), jnp.float32),
            pltpu.SemaphoreType.DMA,
            pltpu.SemaphoreType.DMA,
            pltpu.SemaphoreType.DMA,
            pltpu.SemaphoreType.DMA,
            pltpu.SemaphoreType.DMA,
            pltpu.SemaphoreType.DMA,
            pltpu.SemaphoreType.DMA,
            pltpu.SemaphoreType.DMA,
        ],
        compiler_params=_sc_params(),
    )
    return k(as16, ad16, src_a, dst_a)


def _scA2_body(dnp_hbm, rcp_hbm, inv_hbm, d0_v, d1_v, rcp_v, inv_v):
    c = lax.axis_index("c")
    s = lax.axis_index("s")
    base = (s * NC + c) * RP2
    pltpu.sync_copy(dnp_hbm.at[0, pl.ds(base, RP2)], d0_v)
    pltpu.sync_copy(dnp_hbm.at[1, pl.ds(base, RP2)], d1_v)
    lane = lax.iota(jnp.int32, DH)
    is_den = lane < H

    @pl.loop(0, RP2)
    def _(r):
        drow = d0_v[r] + d1_v[r]
        d0_v[r] = drow
        degv = plsc.load_gather(
            d0_v, [jnp.full((DH,), r, jnp.int32), jnp.full((DH,), H, jnp.int32)])
        degc = jnp.maximum(degv, 1.0)
        rcp_v[r] = 1.0 / jnp.where(is_den, drow + 1e-16, degc)
        inv_v[r] = 1.0 / degc

    pltpu.sync_copy(rcp_v, rcp_hbm.at[pl.ds(base, RP2)])
    pltpu.sync_copy(inv_v, inv_hbm.at[pl.ds(base, RP2)])


def _scA2(dnp):
    k = pl.kernel(
        _scA2_body,
        out_type=(_f32((NP, DH)), _f32((NP, DH))),
        mesh=_mesh,
        scratch_types=[
            pltpu.VMEM((RP2, DH), jnp.float32),
            pltpu.VMEM((RP2, DH), jnp.float32),
            pltpu.VMEM((RP2, DH), jnp.float32),
            pltpu.VMEM((RP2, DH), jnp.float32),
        ],
        compiler_params=_sc_params(),
    )
    return k(dnp)


def _scB_body(xws_hbm, ex_hbm, src_hbm, dst_hbm, agg_hbm,
              src_v, dst_v, gx0, gx1, ob0, ob1, eb0, eb1, cob, agg_sh,
              gsx0, gsx1, gse0, gse1, ssc0, ssc1):
    c = lax.axis_index("c")
    s = lax.axis_index("s")
    base = s * RP
    zero = jnp.zeros((DH,), jnp.float32)
    gx = (gx0, gx1)
    ob = (ob0, ob1)
    eb = (eb0, eb1)
    gsx = (gsx0, gsx1)
    gse = (gse0, gse1)
    ssc = (ssc0, ssc1)

    def g_start(b, jj):
        pltpu.async_copy(xws_hbm.at[src_v.at[jj]], gx[b], gsx[b])
        pltpu.async_copy(ex_hbm.at[s, jj], eb[b], gse[b])

    def g_wait(b, jj):
        pltpu.make_async_copy(xws_hbm.at[src_v.at[jj]], gx[b], gsx[b]).wait()
        pltpu.make_async_copy(ex_hbm.at[s, jj], eb[b], gse[b]).wait()

    def s_start(b, jj):
        pltpu.async_copy(ob[b], agg_sh.at[dst_v.at[jj]], ssc[b], add=True)

    def s_wait(b, jj):
        pltpu.make_async_copy(ob[b], agg_sh.at[dst_v.at[jj]], ssc[b]).wait()

    hvecs = [(jnp.full((DH,), h, jnp.int32) + c * 4)[:, None] for h in range(4)]
    _dnums = lax.GatherDimensionNumbers(
        offset_dims=(), collapsed_slice_dims=(0,), start_index_map=(0,))

    def compute(b):
        @pl.loop(0, CH)
        def _(e):
            erow = eb[b][e]
            for h in range(4):
                bc = lax.gather(erow, hvecs[h], _dnums, (1,),
                                mode=lax.GatherScatterMode.PROMISE_IN_BOUNDS)
                ob[b][e, pl.ds(h * DH, DH)] = gx[b][e, pl.ds(h * DH, DH)] * bc

    @pl.loop(0, 64)
    def _(r):
        for q in range(DHALF // DH):
            cob[r, pl.ds(q * DH, DH)] = zero

    @pl.loop(0, RP // 64)
    def _(p):
        pltpu.sync_copy(cob, agg_sh.at[pl.ds(base + p * 64, 64)])
    pltpu.sync_copy(src_hbm.at[c, s], src_v)
    pltpu.sync_copy(dst_hbm.at[s], dst_v)
    plsc.subcore_barrier()

    g_start(0, 0)
    g_start(1, 1)
    for b in (0, 1):
        g_wait(b, b)
        compute(b)
        s_start(b, b)
        g_start(b, b + 2)

    @pl.loop(1, NCH_B // 2)
    def _(p):
        for b in (0, 1):
            jj = 2 * p + b
            g_wait(b, jj)
            s_wait(b, jj - 2)
            compute(b)
            s_start(b, jj)

            @pl.when(jj + 2 < NCH_B)
            def _(b=b, jj=jj):
                g_start(b, jj + 2)

    s_wait(0, NCH_B - 2)
    s_wait(1, NCH_B - 1)
    plsc.subcore_barrier()

    @pl.loop(0, RP // 64)
    def _(p):
        pltpu.sync_copy(agg_sh.at[pl.ds(base + p * 64, 64)], cob)
        pltpu.sync_copy(cob, agg_hbm.at[c, pl.ds(base + p * 64, 64)])


def _scB(xws, exB, srcB, dstB):
    k = pl.kernel(
        _scB_body,
        out_type=_f32((NC, NP, DHALF)),
        mesh=_mesh,
        scratch_types=[
            pltpu.VMEM((NCH_B, CH), jnp.int32),
            pltpu.VMEM((NCH_B, CH), jnp.int32),
            pltpu.VMEM((CH, DHALF), jnp.float32),
            pltpu.VMEM((CH, DHALF), jnp.float32),
            pltpu.VMEM((CH, DHALF), jnp.float32),
            pltpu.VMEM((CH, DHALF), jnp.float32),
            pltpu.VMEM((CH, DH), jnp.float32),
            pltpu.VMEM((CH, DH), jnp.float32),
            pltpu.VMEM((64, DHALF), jnp.float32),
            pltpu.VMEM_SHARED((NP, DHALF), jnp.float32),
            pltpu.SemaphoreType.DMA,
            pltpu.SemaphoreType.DMA,
            pltpu.SemaphoreType.DMA,
            pltpu.SemaphoreType.DMA,
            pltpu.SemaphoreType.DMA,
            pltpu.SemaphoreType.DMA,
        ],
        compiler_params=_sc_params(),
    )
    return k(xws, exB, srcB, dstB)


def _scC_body(st_hbm, inv_hbm, dw_hbm, pos_hbm, src_hbm, dst_hbm, att_hbm,
              src_v, dst_v, inv_v, cur_v, bias_v, nb_v, z_v, dw_v,
              g0, g1, g2, g3, cur_sh, nxt_sh,
              gs0, gs1, gs2, gs3, ss0, ss1, ss2, ss3):
    c = lax.axis_index("c")
    s = lax.axis_index("s")
    base = s * RP
    zero = jnp.zeros((DH,), jnp.float32)
    lane = lax.iota(jnp.int32, DH)
    m8 = lane == H
    g = (g0, g1, g2, g3)
    gs = (gs0, gs1, gs2, gs3)
    ss = (ss0, ss1, ss2, ss3)

    def g_start(b, jj):
        pltpu.async_copy(cur_sh.at[src_v.at[jj]], g[b], gs[b])

    def g_wait(b, jj):
        pltpu.make_async_copy(cur_sh.at[src_v.at[jj]], g[b], gs[b]).wait()

    def s_start(b, jj):
        pltpu.async_copy(g[b], nxt_sh.at[dst_v.at[jj]], ss[b], add=True)

    def s_wait(b, jj):
        pltpu.make_async_copy(g[b], nxt_sh.at[dst_v.at[jj]], ss[b]).wait()

    pltpu.sync_copy(src_hbm.at[s], src_v)
    pltpu.sync_copy(dst_hbm.at[s], dst_v)
    pltpu.sync_copy(inv_hbm.at[pl.ds(base, RP)], inv_v)
    pltpu.sync_copy(st_hbm.at[pl.ds(base, RP)], cur_v)
    pltpu.sync_copy(cur_v, cur_sh.at[pl.ds(base, RP)])
    pltpu.sync_copy(dw_hbm, dw_v)

    @pl.loop(0, RP)
    def _(r):
        z_v[r] = zero
        bias_v[r] = zero
    pltpu.sync_copy(z_v, nxt_sh.at[pl.ds(base, RP)])
    plsc.subcore_barrier()

    @pl.loop(0, NUM_HOPS)
    def _(hop):
        g_start(0, 0)
        g_start(1, 1)
        g_wait(0, 0)
        s_start(0, 0)
        g_start(2, 2)
        g_wait(1, 1)
        s_start(1, 1)
        g_start(3, 3)
        g_wait(2, 2)
        s_start(2, 2)
        s_wait(0, 0)
        g_start(0, 4)
        g_wait(3, 3)
        s_start(3, 3)
        s_wait(1, 1)
        g_start(1, 5)

        @pl.loop(1, NCH_B // 4)
        def _(p):
            for b in range(4):
                jj = 4 * p + b
                g_wait(b, jj)
                s_start(b, jj)
                b2 = (b + 2) % 4
                s_wait(b2, jj - 2)

                @pl.when(jj + 2 < NCH_B)
                def _(b2=b2, jj=jj):
                    g_start(b2, jj + 2)

        s_wait(2, NCH_B - 2)
        s_wait(3, NCH_B - 1)
        plsc.subcore_barrier()
        pltpu.sync_copy(nxt_sh.at[pl.ds(base, RP)], nb_v)
        dwrow = dw_v[hop]

        @pl.loop(0, RP)
        def _(r):
            srow = nb_v[r]
            crow = cur_v[r]
            fixed = jnp.where(m8, jnp.minimum(crow + srow, 1.0),
                              srow * inv_v[r])
            bias_v[r] = bias_v[r] + dwrow * fixed
            cur_v[r] = fixed
            nb_v[r] = fixed

        pltpu.sync_copy(nb_v, cur_sh.at[pl.ds(base, RP)])
        pltpu.sync_copy(z_v, nxt_sh.at[pl.ds(base, RP)])
        plsc.subcore_barrier()

    @pl.when(c == 0)
    def _():
        pltpu.sync_copy(pos_hbm.at[pl.ds(base, RP)], z_v)

        @pl.loop(0, RP)
        def _(r):
            nb_v[r] = bias_v[r] + z_v[r] + jnp.where(m8, cur_v[r], 0.0)

        pltpu.sync_copy(nb_v, att_hbm.at[pl.ds(base, RP)])


def _scC(state0, inv16, dw16, pos16, src_c, dst_c):
    k = pl.kernel(
        _scC_body,
        out_type=_f32((NP, DH)),
        mesh=_mesh,
        scratch_types=[
            pltpu.VMEM((NCH_B, CH), jnp.int32),
            pltpu.VMEM((NCH_B, CH), jnp.int32),
            pltpu.VMEM((RP, DH), jnp.float32),
            pltpu.VMEM((RP, DH), jnp.float32),
            pltpu.VMEM((RP, DH), jnp.float32),
            pltpu.VMEM((RP, DH), jnp.float32),
            pltpu.VMEM((RP, DH), jnp.float32),
            pltpu.VMEM((NUM_HOPS, DH), jnp.float32),
            pltpu.VMEM((CH, DH), jnp.float32),
            pltpu.VMEM((CH, DH), jnp.float32),
            pltpu.VMEM((CH, DH), jnp.float32),
            pltpu.VMEM((CH, DH), jnp.float32),
            pltpu.VMEM_SHARED((NP, DH), jnp.float32),
            pltpu.VMEM_SHARED((NP, DH), jnp.float32),
            pltpu.SemaphoreType.DMA,
            pltpu.SemaphoreType.DMA,
            pltpu.SemaphoreType.DMA,
            pltpu.SemaphoreType.DMA,
            pltpu.SemaphoreType.DMA,
            pltpu.SemaphoreType.DMA,
            pltpu.SemaphoreType.DMA,
            pltpu.SemaphoreType.DMA,
        ],
        compiler_params=_sc_params(),
    )
    return k(state0, inv16, dw16, pos16, src_c, dst_c)


@jax.jit
def kernel(x, edge_index, position_bias, W_gat, a_src, a_dst, W_fuse, b_fuse,
           diffusion_weight):
    f32 = jnp.float32
    G = jnp.repeat(jnp.eye(H, dtype=f32), DH, axis=0)
    a_s16 = jnp.pad(G * a_src.reshape(-1)[:, None], ((0, 0), (0, H)))
    a_d16 = jnp.pad(G * a_dst.reshape(-1)[:, None], ((0, 0), (0, H)))
    m16 = jnp.pad(G / DH, ((0, 0), (0, H)))
    dw16 = jnp.pad(diffusion_weight.astype(f32), ((0, 0), (0, H)))
    pos16 = jnp.pad(position_bias.astype(f32), ((0, NP - N), (0, H)))
    g8 = jnp.repeat(jnp.eye(H, dtype=f32), DH, axis=1)
    r_lo = jnp.pad(g8[:, :DHALF], ((0, H), (0, 0)))
    r_hi = jnp.pad(g8[:, DHALF:], ((0, H), (0, 0)))
    src_a = edge_index[0].reshape(NC, NS, NCH_A, CH)
    dst_a = edge_index[1].reshape(NC, NS, NCH_A, CH)
    src_t = edge_index[0].reshape(NS, NCH_B, CH)
    dst_t = edge_index[1].reshape(NS, NCH_B, CH)
    src_b = jnp.stack([src_t, src_t + N])

    xw, as16, ad16 = _tc1(x, W_gat, a_s16, a_d16)
    as16 = jnp.pad(as16, ((0, NP - N), (0, 0)))
    ad16 = jnp.pad(ad16, ((0, NP - N), (0, 0)))
    xws = jnp.concatenate([xw[:, :DHALF], xw[:, DHALF:]], axis=0)
    ex16, dnp = _scA(as16, ad16, src_a, dst_a)
    rcp16, inv16 = _scA2(dnp)
    exB = ex16.reshape(NS, NCH_B, CH, DH)
    aggp = _scB(xws, exB, src_b, dst_t)
    state0 = _tc2(x, aggp[0, :N], aggp[1, :N], rcp16[:N], r_lo, r_hi,
                  W_fuse[:D], W_fuse[D:D + DHALF], W_fuse[D + DHALF:],
                  b_fuse.reshape(1, D), m16)
    state0 = jnp.pad(state0, ((0, NP - N), (0, 0)))
    att16 = _scC(state0, inv16, dw16, pos16, src_t, dst_t)
    return (att16[:N, :H], att16[:N, H])
